# Initial kernel scaffold; baseline (speedup 1.0000x reference)
import functools

import jax
import jax.numpy as jnp
from jax import lax
from jax.experimental import pallas as pl
from jax.experimental.pallas import tpu as pltpu
from jax.experimental.pallas import tpu_sc as plsc

DIM = 128
AUG = 144
SLOPE = 0.1
NC = 2
NS = 16
NW = NC * NS
BLK = 128


def _embed_body(x_ref, w_ref, b_ref, a_ref, emb_ref, s1_ref, s2_ref):
    n = x_ref.shape[0]
    emb = jnp.dot(x_ref[...], w_ref[...], preferred_element_type=jnp.float32)
    emb = emb + b_ref[...][None, :]
    emb_ref[...] = jnp.zeros_like(emb_ref)
    emb_ref[0:n, 0:DIM] = emb
    emb_ref[0:n, DIM:DIM + 1] = jnp.ones((n, 1), jnp.float32)
    a1 = a_ref[0:DIM, 0]
    a2 = a_ref[DIM:2 * DIM, 0]
    s1_ref[...] = jnp.zeros_like(s1_ref)
    s2_ref[...] = jnp.zeros_like(s2_ref)
    s1_ref[0:n] = jnp.sum(emb * a1[None, :], axis=1)
    s2_ref[0:n] = jnp.sum(emb * a2[None, :], axis=1)


def _combine_body(p_ref, o_ref):
    n = o_ref.shape[0]
    p = p_ref[0] + p_ref[1]
    num = p[0:n, 0:DIM]
    den = p[0:n, DIM:DIM + 1]
    o_ref[...] = num / den


def _make_agg(n_pad, nb, acc_rows):
    mesh = plsc.VectorSubcoreMesh(core_axis_name="c", subcore_axis_name="s")
    zero_rows = acc_rows // NS
    out_rows = n_pad // NS

    @functools.partial(
        pl.kernel,
        out_type=jax.ShapeDtypeStruct((NC, n_pad, AUG), jnp.float32),
        mesh=mesh,
        scratch_types=[
            pltpu.VMEM((nb, BLK), jnp.int32),
            pltpu.VMEM((nb, BLK), jnp.int32),
            pltpu.VMEM((n_pad,), jnp.float32),
            pltpu.VMEM((n_pad,), jnp.float32),
            pltpu.VMEM((BLK, AUG), jnp.float32),
            pltpu.VMEM((BLK,), jnp.float32),
            pltpu.VMEM_SHARED((acc_rows, AUG), jnp.float32),
            pltpu.SemaphoreType.DMA,
        ],
    )
    def agg(emb_hbm, src_hbm, dst_hbm, s1_hbm, s2_hbm, out_hbm,
            src_v, dst_v, s1_v, s2_v, rows_v, w_v, acc_sh, sem):
        cid = lax.axis_index("c")
        sid = lax.axis_index("s")
        wid = sid * NC + cid

        @pl.loop(0, BLK)
        def _zrow(r):
            for g in range(AUG // 16):
                rows_v[r, pl.ds(g * 16, 16)] = jnp.zeros((16,), jnp.float32)

        for i in range(zero_rows // BLK):
            pltpu.sync_copy(
                rows_v, acc_sh.at[pl.ds(sid * zero_rows + i * BLK, BLK)])

        pltpu.sync_copy(src_hbm.at[wid], src_v)
        pltpu.sync_copy(dst_hbm.at[wid], dst_v)
        pltpu.sync_copy(s1_hbm, s1_v)
        pltpu.sync_copy(s2_hbm, s2_v)
        plsc.subcore_barrier()

        @pl.loop(0, nb)
        def _blk(j):
            pltpu.async_copy(emb_hbm.at[dst_v.at[j]], rows_v, sem).wait()
            for g in range(BLK // 16):
                sl = pl.ds(g * 16, 16)
                x = (plsc.load_gather(s1_v, [src_v[j, sl]])
                     + plsc.load_gather(s2_v, [dst_v[j, sl]]))
                lr = jnp.where(x > 0.0, x, x * SLOPE)
                w_v[sl] = jnp.exp(lr)
            for r in range(BLK):
                w = w_v[r]
                for g in range(AUG // 16):
                    sl = pl.ds(g * 16, 16)
                    rows_v[r, sl] = rows_v[r, sl] * w
            pltpu.sync_copy(rows_v, acc_sh.at[src_v.at[j]], add=True)

        plsc.subcore_barrier()
        pltpu.sync_copy(acc_sh.at[pl.ds(sid * out_rows, out_rows)],
                        out_hbm.at[cid, pl.ds(sid * out_rows, out_rows)])

    return agg


def kernel(nodes, edge_index, local_features, W, b, a):
    n = local_features.shape[0]
    e = edge_index.shape[1]
    n_pad = ((n + 1 + 15) // 16) * 16
    n_pad = ((n_pad + NS - 1) // NS) * NS
    e_tot = e + n
    epb = NW * BLK
    nb = (e_tot + epb - 1) // epb
    e_pad = nb * epb
    acc_rows = ((n_pad + NS * BLK - 1) // (NS * BLK)) * (NS * BLK)

    nodes_i = nodes.astype(jnp.int32)
    src = jnp.concatenate([
        edge_index[0].astype(jnp.int32), nodes_i,
        jnp.full((e_pad - e_tot,), n, jnp.int32)])
    dst = jnp.concatenate([
        edge_index[1].astype(jnp.int32), nodes_i,
        jnp.zeros((e_pad - e_tot,), jnp.int32)])
    src3 = src.reshape(NW, nb, BLK)
    dst3 = dst.reshape(NW, nb, BLK)

    emb_aug, s1, s2 = pl.pallas_call(
        _embed_body,
        out_shape=(
            jax.ShapeDtypeStruct((n, AUG), jnp.float32),
            jax.ShapeDtypeStruct((n_pad,), jnp.float32),
            jax.ShapeDtypeStruct((n_pad,), jnp.float32),
        ),
    )(local_features, W, b, a)

    parts = _make_agg(n_pad, nb, acc_rows)(emb_aug, src3, dst3, s1, s2)

    out = pl.pallas_call(
        _combine_body,
        out_shape=jax.ShapeDtypeStruct((n, DIM), jnp.float32),
    )(parts)
    return out


# speedup vs baseline: 7.9338x; 7.9338x over previous
import functools

import jax
import jax.numpy as jnp
from jax import lax
from jax.experimental import pallas as pl
from jax.experimental.pallas import tpu as pltpu
from jax.experimental.pallas import tpu_sc as plsc

DIM = 128
AUG = 144
SLOPE = 0.1
NC = 2
NS = 16
NW = NC * NS
BLK = 128
CHUNK = 3


def _embed_body(x_ref, w_ref, b_ref, a_ref, emb_ref, s1_ref, s2_ref):
    n = x_ref.shape[0]
    emb = jnp.dot(x_ref[...], w_ref[...], preferred_element_type=jnp.float32)
    emb = emb + b_ref[...][None, :]
    emb_ref[...] = jnp.zeros_like(emb_ref)
    emb_ref[0:n, 0:DIM] = emb
    emb_ref[0:n, DIM:DIM + 1] = jnp.ones((n, 1), jnp.float32)
    a1 = a_ref[0:DIM, 0]
    a2 = a_ref[DIM:2 * DIM, 0]
    s1_ref[...] = jnp.zeros_like(s1_ref)
    s2_ref[...] = jnp.zeros_like(s2_ref)
    s1_ref[0:n] = jnp.sum(emb * a1[None, :], axis=1)
    s2_ref[0:n] = jnp.sum(emb * a2[None, :], axis=1)


def _combine_body(p_ref, o_ref):
    n = o_ref.shape[0]
    p = p_ref[0] + p_ref[1]
    o_ref[...] = p[0:n, 0:DIM] / p[0:n, DIM:DIM + 1]


def _make_agg(n_pad, nb, acc_rows):
    mesh = plsc.VectorSubcoreMesh(core_axis_name="c", subcore_axis_name="s")
    zero_rows = acc_rows // NS
    out_rows = n_pad // NS

    @functools.partial(
        pl.kernel,
        out_type=jax.ShapeDtypeStruct((NC, n_pad, AUG), jnp.float32),
        mesh=mesh,
        compiler_params=pltpu.CompilerParams(
            use_tc_tiling_on_sc=False, needs_layout_passes=False),
        scratch_types=[
            pltpu.VMEM((CHUNK, BLK), jnp.int32),
            pltpu.VMEM((CHUNK, BLK), jnp.int32),
            pltpu.VMEM((n_pad,), jnp.float32),
            pltpu.VMEM((n_pad,), jnp.float32),
            pltpu.VMEM((BLK, AUG), jnp.float32),
            pltpu.VMEM((BLK,), jnp.float32),
            pltpu.VMEM_SHARED((acc_rows, AUG), jnp.float32),
            pltpu.SemaphoreType.DMA,
        ],
    )
    def agg(emb_hbm, src_hbm, dst_hbm, s1_hbm, s2_hbm, out_hbm,
            src_v, dst_v, s1_v, s2_v, rows_v, w_v, acc_sh, sem):
        cid = lax.axis_index("c")
        sid = lax.axis_index("s")
        wid = sid * NC + cid

        @pl.loop(0, BLK)
        def _zrow(r):
            for g in range(AUG // 16):
                rows_v[r, pl.ds(g * 16, 16)] = jnp.zeros((16,), jnp.float32)
        for i in range(zero_rows // BLK):
            pltpu.sync_copy(
                rows_v, acc_sh.at[pl.ds(sid * zero_rows + i * BLK, BLK)])
        rem = zero_rows % BLK
        if rem:
            pltpu.sync_copy(
                rows_v.at[pl.ds(0, rem)],
                acc_sh.at[pl.ds(sid * zero_rows + (zero_rows - rem), rem)])

        pltpu.sync_copy(s1_hbm, s1_v)
        pltpu.sync_copy(s2_hbm, s2_v)
        plsc.subcore_barrier()

        @pl.loop(0, nb)
        def _blk(j):
            c = j % CHUNK

            @pl.when(c == 0)
            def _stage():
                pltpu.sync_copy(src_hbm.at[wid, pl.ds(j, CHUNK)], src_v)
                pltpu.sync_copy(dst_hbm.at[wid, pl.ds(j, CHUNK)], dst_v)

            pltpu.async_copy(emb_hbm.at[dst_v.at[c]], rows_v, sem).wait()
            for g in range(BLK // 16):
                sl = pl.ds(g * 16, 16)
                x = (plsc.load_gather(s1_v, [src_v[c, sl]])
                     + plsc.load_gather(s2_v, [dst_v[c, sl]]))
                lr = jnp.where(x > 0.0, x, x * SLOPE)
                w_v[sl] = jnp.exp(lr)
            for g16 in range(BLK // 16):
                wv = w_v[pl.ds(g16 * 16, 16)]
                for k in range(16):
                    r = g16 * 16 + k
                    w = wv[k]
                    for g in range(AUG // 16):
                        sl = pl.ds(g * 16, 16)
                        rows_v[r, sl] = rows_v[r, sl] * w
            pltpu.sync_copy(rows_v, acc_sh.at[src_v.at[c]], add=True)

        plsc.subcore_barrier()
        pltpu.sync_copy(acc_sh.at[pl.ds(sid * out_rows, out_rows)],
                        out_hbm.at[cid, pl.ds(sid * out_rows, out_rows)])

    return agg


def kernel(nodes, edge_index, local_features, W, b, a):
    n = local_features.shape[0]
    e = edge_index.shape[1]
    n_pad = ((n + 1 + 127) // 128) * 128
    e_tot = e + n
    epb = NW * BLK * CHUNK
    nb = CHUNK * ((e_tot + epb - 1) // epb)
    e_pad = nb * NW * BLK
    acc_rows = n_pad

    nodes_i = nodes.astype(jnp.int32)
    src = jnp.concatenate([
        edge_index[0].astype(jnp.int32), nodes_i,
        jnp.full((e_pad - e_tot,), n, jnp.int32)])
    dst = jnp.concatenate([
        edge_index[1].astype(jnp.int32), nodes_i,
        jnp.zeros((e_pad - e_tot,), jnp.int32)])
    src3 = src.reshape(NW, nb, BLK)
    dst3 = dst.reshape(NW, nb, BLK)

    emb_aug, s1, s2 = pl.pallas_call(
        _embed_body,
        out_shape=(
            jax.ShapeDtypeStruct((n, AUG), jnp.float32),
            jax.ShapeDtypeStruct((n_pad,), jnp.float32),
            jax.ShapeDtypeStruct((n_pad,), jnp.float32),
        ),
    )(local_features, W, b, a)

    parts = _make_agg(n_pad, nb, acc_rows)(emb_aug, src3, dst3, s1, s2)

    out = pl.pallas_call(
        _combine_body,
        out_shape=jax.ShapeDtypeStruct((n, DIM), jnp.float32),
    )(parts)
    return out


# speedup vs baseline: 10.9292x; 1.3775x over previous
import functools

import jax
import jax.numpy as jnp
from jax import lax
from jax.experimental import pallas as pl
from jax.experimental.pallas import tpu as pltpu
from jax.experimental.pallas import tpu_sc as plsc

DIM = 128
AUG = 144
SLOPE = 0.1
NC = 2
NS = 16
NW = NC * NS
BLK = 96
IDXROWS = 8


def _embed_body(x_ref, w_ref, b_ref, a_ref, emb_ref, sp_ref):
    n = x_ref.shape[0]
    emb = jnp.dot(x_ref[...], w_ref[...], preferred_element_type=jnp.float32)
    emb = emb + b_ref[...][None, :]
    emb_ref[...] = jnp.zeros_like(emb_ref)
    emb_ref[0:n, 0:DIM] = emb
    emb_ref[0:n, DIM:DIM + 1] = jnp.ones((n, 1), jnp.float32)
    a1 = a_ref[0:DIM, 0]
    a2 = a_ref[DIM:2 * DIM, 0]
    s1 = jnp.sum(emb * a1[None, :], axis=1)
    s2 = jnp.sum(emb * a2[None, :], axis=1)
    u1 = lax.bitcast_convert_type(s1.astype(jnp.bfloat16), jnp.uint16)
    u2 = lax.bitcast_convert_type(s2.astype(jnp.bfloat16), jnp.uint16)
    packed = u1.astype(jnp.uint32) | (u2.astype(jnp.uint32) << 16)
    sp_ref[...] = jnp.zeros_like(sp_ref)
    sp_ref[0:n] = lax.bitcast_convert_type(packed, jnp.float32)


def _combine_body(p_ref, o_ref):
    n = o_ref.shape[0]
    p = p_ref[0] + p_ref[1]
    o_ref[...] = p[0:n, 0:DIM] / p[0:n, DIM:DIM + 1]


def _make_agg(n_pad, nb):
    mesh = plsc.VectorSubcoreMesh(core_axis_name="c", subcore_axis_name="s")
    acc_rows = n_pad
    zero_rows = acc_rows // NS
    out_rows = n_pad // NS
    hi_mask = jnp.int32(-65536)

    @functools.partial(
        pl.kernel,
        out_type=jax.ShapeDtypeStruct((NC, n_pad, AUG), jnp.float32),
        mesh=mesh,
        compiler_params=pltpu.CompilerParams(
            use_tc_tiling_on_sc=False, needs_layout_passes=False),
        scratch_types=[
            pltpu.VMEM((IDXROWS, BLK), jnp.int32),
            pltpu.VMEM((IDXROWS, BLK), jnp.int32),
            pltpu.VMEM((n_pad,), jnp.float32),
            pltpu.VMEM((2, BLK, AUG), jnp.float32),
            pltpu.VMEM((BLK,), jnp.float32),
            pltpu.VMEM_SHARED((acc_rows, AUG), jnp.float32),
            pltpu.SemaphoreType.DMA,
            pltpu.SemaphoreType.DMA,
            pltpu.SemaphoreType.DMA,
            pltpu.SemaphoreType.DMA,
        ],
    )
    def agg(emb_hbm, src_hbm, dst_hbm, sp_hbm, out_hbm,
            src_v, dst_v, sp_v, rows_v, w_v, acc_sh,
            gsem0, gsem1, ssem0, ssem1):
        cid = lax.axis_index("c")
        sid = lax.axis_index("s")
        wid = sid * NC + cid
        gsems = (gsem0, gsem1)
        ssems = (ssem0, ssem1)

        def gather(i_row, buf, sem):
            return pltpu.async_copy(
                emb_hbm.at[dst_v.at[i_row]], rows_v.at[buf], sem)

        def scatter(i_row, buf, sem):
            return pltpu.async_copy(
                rows_v.at[buf], acc_sh.at[src_v.at[i_row]], sem, add=True)

        def wait_gather(i_row, buf, sem):
            pltpu.make_async_copy(
                emb_hbm.at[dst_v.at[i_row]], rows_v.at[buf], sem).wait()

        def wait_scatter(i_row, buf, sem):
            pltpu.make_async_copy(
                rows_v.at[buf], acc_sh.at[src_v.at[i_row]], sem).wait()

        @pl.loop(0, BLK)
        def _zrow(r):
            for g in range(AUG // 16):
                rows_v[0, r, pl.ds(g * 16, 16)] = jnp.zeros((16,),
                                                            jnp.float32)
        for i in range(zero_rows // BLK):
            pltpu.sync_copy(
                rows_v.at[0],
                acc_sh.at[pl.ds(sid * zero_rows + i * BLK, BLK)])
        rem = zero_rows % BLK
        if rem:
            pltpu.sync_copy(
                rows_v.at[0, pl.ds(0, rem)],
                acc_sh.at[pl.ds(sid * zero_rows + (zero_rows - rem), rem)])

        pltpu.sync_copy(sp_hbm, sp_v)
        pltpu.sync_copy(src_hbm.at[wid, pl.ds(0, IDXROWS)], src_v)
        pltpu.sync_copy(dst_hbm.at[wid, pl.ds(0, IDXROWS)], dst_v)
        plsc.subcore_barrier()

        gather(0, 0, gsems[0])

        @pl.loop(0, nb)
        def _blk(i):
            p = i % 2
            c = i % IDXROWS
            half = IDXROWS // 2

            @pl.when(i > 0)
            def _drain_prev():
                for q in range(2):
                    @pl.when(p == q)
                    def _w():
                        wait_scatter((i - 1) % IDXROWS, 1 - q, ssems[1 - q])

            @pl.when(jnp.logical_and(i % half == 0, i + half < nb))
            def _stage():
                tgt = ((i + half) % IDXROWS) // half
                pltpu.sync_copy(
                    src_hbm.at[wid, pl.ds(i + half, half)],
                    src_v.at[pl.ds(tgt * half, half)])
                pltpu.sync_copy(
                    dst_hbm.at[wid, pl.ds(i + half, half)],
                    dst_v.at[pl.ds(tgt * half, half)])

            @pl.when(i + 1 < nb)
            def _prefetch():
                for q in range(2):
                    @pl.when(p == q)
                    def _g():
                        gather((i + 1) % IDXROWS, 1 - q, gsems[1 - q])

            for q in range(2):
                @pl.when(p == q)
                def _wg():
                    wait_gather(c, q, gsems[q])

            for g in range(BLK // 16):
                sl = pl.ds(g * 16, 16)
                pk_s = plsc.load_gather(sp_v, [src_v[c, sl]])
                pk_d = plsc.load_gather(sp_v, [dst_v[c, sl]])
                s1 = plsc.bitcast(
                    plsc.bitcast(pk_s, jnp.int32) << 16, jnp.float32)
                s2 = plsc.bitcast(
                    plsc.bitcast(pk_d, jnp.int32) & hi_mask, jnp.float32)
                x = s1 + s2
                lr = jnp.where(x > 0.0, x, x * SLOPE)
                w_v[sl] = jnp.exp(lr)
            for g16 in range(BLK // 16):
                wv = w_v[pl.ds(g16 * 16, 16)]
                for k in range(16):
                    r = g16 * 16 + k
                    w = wv[k]
                    for g in range(AUG // 16):
                        sl = pl.ds(g * 16, 16)
                        rows_v[p, r, sl] = rows_v[p, r, sl] * w

            for q in range(2):
                @pl.when(p == q)
                def _s():
                    scatter(c, q, ssems[q])

        qlast = (nb - 1) % 2
        wait_scatter((nb - 1) % IDXROWS, qlast, ssems[qlast])

        plsc.subcore_barrier()
        pltpu.sync_copy(acc_sh.at[pl.ds(sid * out_rows, out_rows)],
                        out_hbm.at[cid, pl.ds(sid * out_rows, out_rows)])

    return agg


def kernel(nodes, edge_index, local_features, W, b, a):
    n = local_features.shape[0]
    e = edge_index.shape[1]
    n_pad = ((n + 1 + 127) // 128) * 128
    e_tot = e + n
    epb = NW * BLK
    nb = 2 * ((e_tot + 2 * epb - 1) // (2 * epb))
    e_pad = nb * epb

    nodes_i = nodes.astype(jnp.int32)
    src = jnp.concatenate([
        edge_index[0].astype(jnp.int32), nodes_i,
        jnp.full((e_pad - e_tot,), n, jnp.int32)])
    dst = jnp.concatenate([
        edge_index[1].astype(jnp.int32), nodes_i,
        jnp.zeros((e_pad - e_tot,), jnp.int32)])
    src3 = src.reshape(NW, nb, BLK)
    dst3 = dst.reshape(NW, nb, BLK)

    emb_aug, sp = pl.pallas_call(
        _embed_body,
        out_shape=(
            jax.ShapeDtypeStruct((n, AUG), jnp.float32),
            jax.ShapeDtypeStruct((n_pad,), jnp.float32),
        ),
    )(local_features, W, b, a)

    parts = _make_agg(n_pad, nb)(emb_aug, src3, dst3, sp)

    out = pl.pallas_call(
        _combine_body,
        out_shape=jax.ShapeDtypeStruct((n, DIM), jnp.float32),
    )(parts)
    return out


# speedup vs baseline: 11.4955x; 1.0518x over previous
import functools

import jax
import jax.numpy as jnp
from jax import lax
from jax.experimental import pallas as pl
from jax.experimental.pallas import tpu as pltpu
from jax.experimental.pallas import tpu_sc as plsc

DIM = 128
AUG = 144
SLOPE = 0.1
NC = 2
NS = 16
NW = NC * NS
BLK = 96
IDXROWS = 8


def _embed_body(x_ref, w_ref, b_ref, a_ref, emb_ref, sp_ref):
    n = x_ref.shape[0]
    emb = jnp.dot(x_ref[...], w_ref[...], preferred_element_type=jnp.float32)
    emb = emb + b_ref[...][None, :]
    emb_ref[...] = jnp.zeros_like(emb_ref)
    emb_ref[0:n, 0:DIM] = emb
    emb_ref[0:n, DIM:DIM + 1] = jnp.ones((n, 1), jnp.float32)
    a1 = a_ref[0:DIM, 0]
    a2 = a_ref[DIM:2 * DIM, 0]
    s1 = jnp.sum(emb * a1[None, :], axis=1)
    s2 = jnp.sum(emb * a2[None, :], axis=1)
    u1 = lax.bitcast_convert_type(s1.astype(jnp.bfloat16), jnp.uint16)
    u2 = lax.bitcast_convert_type(s2.astype(jnp.bfloat16), jnp.uint16)
    packed = u1.astype(jnp.uint32) | (u2.astype(jnp.uint32) << 16)
    sp_ref[...] = jnp.zeros_like(sp_ref)
    sp_ref[0:n] = lax.bitcast_convert_type(packed, jnp.float32)


def _combine_body(p_ref, o_ref):
    n = o_ref.shape[0]
    p = p_ref[0] + p_ref[1]
    o_ref[...] = p[0:n, 0:DIM] / p[0:n, DIM:DIM + 1]


def _make_agg(n_pad, nb):
    mesh = plsc.VectorSubcoreMesh(core_axis_name="c", subcore_axis_name="s")
    acc_rows = n_pad
    zero_rows = acc_rows // NS
    out_rows = n_pad // NS
    hi_mask = jnp.int32(-65536)

    @functools.partial(
        pl.kernel,
        out_type=jax.ShapeDtypeStruct((NC, n_pad, AUG), jnp.float32),
        mesh=mesh,
        compiler_params=pltpu.CompilerParams(
            use_tc_tiling_on_sc=False, needs_layout_passes=False),
        scratch_types=[
            pltpu.VMEM((IDXROWS, BLK), jnp.int32),
            pltpu.VMEM((IDXROWS, BLK), jnp.int32),
            pltpu.VMEM((n_pad,), jnp.float32),
            pltpu.VMEM((2, BLK, AUG), jnp.float32),
            pltpu.VMEM((BLK,), jnp.float32),
            pltpu.VMEM_SHARED((acc_rows, AUG), jnp.float32),
            pltpu.SemaphoreType.DMA,
            pltpu.SemaphoreType.DMA,
            pltpu.SemaphoreType.DMA,
            pltpu.SemaphoreType.DMA,
            pltpu.SemaphoreType.DMA,
        ],
    )
    def agg(emb_hbm, src_hbm, dst_hbm, sp_hbm, out_hbm,
            src_v, dst_v, sp_v, rows_v, w_v, acc_sh,
            gsem0, gsem1, ssem0, ssem1, isem):
        cid = lax.axis_index("c")
        sid = lax.axis_index("s")
        wid = sid * NC + cid
        gsems = (gsem0, gsem1)
        ssems = (ssem0, ssem1)

        def gather(i_row, buf, sem):
            return pltpu.async_copy(
                emb_hbm.at[dst_v.at[i_row]], rows_v.at[buf], sem)

        def scatter(i_row, buf, sem):
            return pltpu.async_copy(
                rows_v.at[buf], acc_sh.at[src_v.at[i_row]], sem, add=True)

        def wait_gather(i_row, buf, sem):
            pltpu.make_async_copy(
                emb_hbm.at[dst_v.at[i_row]], rows_v.at[buf], sem).wait()

        def wait_scatter(i_row, buf, sem):
            pltpu.make_async_copy(
                rows_v.at[buf], acc_sh.at[src_v.at[i_row]], sem).wait()

        @pl.loop(0, BLK)
        def _zrow(r):
            for g in range(AUG // 16):
                rows_v[0, r, pl.ds(g * 16, 16)] = jnp.zeros((16,),
                                                            jnp.float32)
        for i in range(zero_rows // BLK):
            pltpu.sync_copy(
                rows_v.at[0],
                acc_sh.at[pl.ds(sid * zero_rows + i * BLK, BLK)])
        rem = zero_rows % BLK
        if rem:
            pltpu.sync_copy(
                rows_v.at[0, pl.ds(0, rem)],
                acc_sh.at[pl.ds(sid * zero_rows + (zero_rows - rem), rem)])

        pltpu.sync_copy(sp_hbm, sp_v)
        pltpu.sync_copy(src_hbm.at[wid, pl.ds(0, IDXROWS)], src_v)
        pltpu.sync_copy(dst_hbm.at[wid, pl.ds(0, IDXROWS)], dst_v)
        plsc.subcore_barrier()

        gather(0, 0, gsems[0])

        @pl.loop(0, nb)
        def _blk(i):
            p = i % 2
            c = i % IDXROWS
            half = IDXROWS // 2

            @pl.when(i > 0)
            def _drain_prev():
                for q in range(2):
                    @pl.when(p == q)
                    def _w():
                        wait_scatter((i - 1) % IDXROWS, 1 - q, ssems[1 - q])

            @pl.when(jnp.logical_and(i % half == 0, i + half < nb))
            def _stage():
                tgt = ((i + half) % IDXROWS) // half
                pltpu.async_copy(
                    src_hbm.at[wid, pl.ds(i + half, half)],
                    src_v.at[pl.ds(tgt * half, half)], isem)
                pltpu.async_copy(
                    dst_hbm.at[wid, pl.ds(i + half, half)],
                    dst_v.at[pl.ds(tgt * half, half)], isem)

            @pl.when(jnp.logical_and(i % half == half - 1, i + 1 < nb))
            def _stage_wait():
                tgt = ((i + 1) % IDXROWS) // half
                pltpu.make_async_copy(
                    src_hbm.at[wid, pl.ds(i + 1, half)],
                    src_v.at[pl.ds(tgt * half, half)], isem).wait()
                pltpu.make_async_copy(
                    dst_hbm.at[wid, pl.ds(i + 1, half)],
                    dst_v.at[pl.ds(tgt * half, half)], isem).wait()

            @pl.when(i + 1 < nb)
            def _prefetch():
                for q in range(2):
                    @pl.when(p == q)
                    def _g():
                        gather((i + 1) % IDXROWS, 1 - q, gsems[1 - q])

            for q in range(2):
                @pl.when(p == q)
                def _wg():
                    wait_gather(c, q, gsems[q])

            for g in range(BLK // 16):
                sl = pl.ds(g * 16, 16)
                pk_s = plsc.load_gather(sp_v, [src_v[c, sl]])
                pk_d = plsc.load_gather(sp_v, [dst_v[c, sl]])
                s1 = plsc.bitcast(
                    plsc.bitcast(pk_s, jnp.int32) << 16, jnp.float32)
                s2 = plsc.bitcast(
                    plsc.bitcast(pk_d, jnp.int32) & hi_mask, jnp.float32)
                x = s1 + s2
                lr = jnp.where(x > 0.0, x, x * SLOPE)
                w_v[sl] = jnp.exp(lr)
            for g16 in range(BLK // 16):
                wv = w_v[pl.ds(g16 * 16, 16)]
                for k in range(16):
                    r = g16 * 16 + k
                    w = wv[k]
                    for g in range(AUG // 16):
                        sl = pl.ds(g * 16, 16)
                        rows_v[p, r, sl] = rows_v[p, r, sl] * w

            for q in range(2):
                @pl.when(p == q)
                def _s():
                    scatter(c, q, ssems[q])

        qlast = (nb - 1) % 2
        wait_scatter((nb - 1) % IDXROWS, qlast, ssems[qlast])

        plsc.subcore_barrier()
        pltpu.sync_copy(acc_sh.at[pl.ds(sid * out_rows, out_rows)],
                        out_hbm.at[cid, pl.ds(sid * out_rows, out_rows)])

    return agg


def kernel(nodes, edge_index, local_features, W, b, a):
    n = local_features.shape[0]
    e = edge_index.shape[1]
    n_pad = ((n + 1 + 127) // 128) * 128
    e_tot = e + n
    epb = NW * BLK
    nb = 2 * ((e_tot + 2 * epb - 1) // (2 * epb))
    e_pad = nb * epb

    nodes_i = nodes.astype(jnp.int32)
    src = jnp.concatenate([
        edge_index[0].astype(jnp.int32), nodes_i,
        jnp.full((e_pad - e_tot,), n, jnp.int32)])
    dst = jnp.concatenate([
        edge_index[1].astype(jnp.int32), nodes_i,
        jnp.zeros((e_pad - e_tot,), jnp.int32)])
    src3 = src.reshape(NW, nb, BLK)
    dst3 = dst.reshape(NW, nb, BLK)

    emb_aug, sp = pl.pallas_call(
        _embed_body,
        out_shape=(
            jax.ShapeDtypeStruct((n, AUG), jnp.float32),
            jax.ShapeDtypeStruct((n_pad,), jnp.float32),
        ),
    )(local_features, W, b, a)

    parts = _make_agg(n_pad, nb)(emb_aug, src3, dst3, sp)

    out = pl.pallas_call(
        _combine_body,
        out_shape=jax.ShapeDtypeStruct((n, DIM), jnp.float32),
    )(parts)
    return out
